# baseline (speedup 1.0000x reference)
import functools

import jax
import jax.numpy as jnp
from jax import lax
from jax.experimental import pallas as pl
from jax.experimental.pallas import tpu as pltpu
from jax.experimental.pallas import tpu_sc as plsc

L = 16
NC, NS = 2, 16
NW = NC * NS


def _sc_loss(preds_hbm, tgt_hbm, len_hbm, out_hbm, pred_v, tgt_v, len_v, out_v,
             scr_v):
    B = tgt_hbm.shape[0]
    C = pred_v.shape[1]
    wpb = NW // B

    wid = lax.axis_index("s") * NC + lax.axis_index("c")
    b = wid // wpb
    j = wid % wpb

    pltpu.sync_copy(preds_hbm.at[b], pred_v)
    pltpu.sync_copy(tgt_hbm.at[b], tgt_v)
    pltpu.sync_copy(len_hbm, len_v)

    iota = lax.iota(jnp.int32, L)
    scr_v[...] = len_v[...][jnp.full((L,), b)]
    length_b = scr_v[...][0]

    nk = (length_b - j + (wpb - 1)) // wpb

    def per_target(k, acc):
        t = j + wpb * k
        trow = tgt_v[t]
        s_cx, s_cy, s_w, s_h = trow[1], trow[2], trow[3], trow[4]
        t_x1 = jnp.full((L,), s_cx - s_w * 0.5)
        t_y1 = jnp.full((L,), s_cy - s_h * 0.5)
        t_x2 = jnp.full((L,), s_cx + s_w * 0.5)
        t_y2 = jnp.full((L,), s_cy + s_h * 0.5)
        area_t = (t_x2 - t_x1) * (t_y2 - t_y1)

        def per_chunk(c, carry):
            run_max, run_idx, idxs = carry
            px = pred_v[1, c]
            py = pred_v[2, c]
            pw = pred_v[3, c]
            ph = pred_v[4, c]
            hw = pw * 0.5
            hh = ph * 0.5
            px1 = px - hw
            px2 = px + hw
            py1 = py - hh
            py2 = py + hh
            ix1 = jnp.maximum(px1, t_x1)
            iy1 = jnp.maximum(py1, t_y1)
            ix2 = jnp.minimum(px2, t_x2)
            iy2 = jnp.minimum(py2, t_y2)
            inter = jnp.maximum(0.0, ix2 - ix1) * jnp.maximum(0.0, iy2 - iy1)
            area_p = (px2 - px1) * (py2 - py1)
            iou = inter / (area_p + area_t - inter + 1e-06)
            upd = iou > run_max
            run_max = jnp.where(upd, iou, run_max)
            run_idx = jnp.where(upd, idxs, run_idx)
            return run_max, run_idx, idxs + L

        init = (jnp.full((L,), -jnp.inf, jnp.float32), jnp.zeros((L,), jnp.int32), iota)
        run_max, run_idx, _ = lax.fori_loop(0, C, per_chunk, init)

        m = run_max
        for s in (8, 4, 2, 1):
            m = jnp.maximum(m, m[iota ^ s])
        cand = jnp.where(run_max == m, run_idx.astype(jnp.float32),
                         jnp.full((L,), float(C * L), jnp.float32))
        for s in (8, 4, 2, 1):
            cand = jnp.minimum(cand, cand[iota ^ s])
        bestv = jnp.where(m > 0.0, cand, 0.0).astype(jnp.int32)
        scr_v[...] = bestv
        best = scr_v[...][0]

        cb = lax.shift_right_logical(best, 4)
        lane = jnp.full((L,), best & (L - 1))
        loss = jnp.zeros((L,), jnp.float32)
        for r in range(5):
            pv = pred_v[r, cb]
            dv = pv[lane] - jnp.full((L,), trow[r])
            loss = loss + dv * dv
        return acc + loss

    acc = lax.fori_loop(0, nk, per_target, jnp.zeros((L,), jnp.float32))
    out_v[...] = jnp.where(iota == 0, acc, 0.0)
    pltpu.sync_copy(out_v, out_hbm.at[wid])


def kernel(predictions, targets, lengths):
    B, N, F = predictions.shape
    C = N // L
    preds_t = jnp.reshape(predictions.transpose(0, 2, 1), (B, F, C, L))
    tgt_pad = jnp.pad(targets, ((0, 0), (0, 0), (0, L - targets.shape[2])))
    len_pad = jnp.pad(lengths.astype(jnp.int32), (0, L - B))

    mesh = plsc.VectorSubcoreMesh(core_axis_name="c", subcore_axis_name="s",
                                  num_cores=NC, num_subcores=NS)
    run = pl.kernel(
        _sc_loss,
        out_type=jax.ShapeDtypeStruct((NW, L), jnp.float32),
        mesh=mesh,
        compiler_params=pltpu.CompilerParams(use_tc_tiling_on_sc=False),
        scratch_types=[
            pltpu.VMEM((F, C, L), jnp.float32),
            pltpu.VMEM((targets.shape[1], L), jnp.float32),
            pltpu.VMEM((L,), jnp.int32),
            pltpu.VMEM((L,), jnp.float32),
            pltpu.VMEM((L,), jnp.int32),
        ],
    )
    partial = run(preds_t, tgt_pad, len_pad)
    return jnp.sum(partial) / B


# speedup vs baseline: 1.2168x; 1.2168x over previous
import functools

import jax
import jax.numpy as jnp
from jax import lax
from jax.experimental import pallas as pl
from jax.experimental.pallas import tpu as pltpu
from jax.experimental.pallas import tpu_sc as plsc

L = 16
NC, NS = 2, 16
NW = NC * NS


def _sc_loss(preds_hbm, tgt_hbm, len_hbm, out_hbm, pred_v, tgt_v, len_v, out_v,
             scr_v):
    B = tgt_hbm.shape[0]
    C = pred_v.shape[1]
    wpb = NW // B

    wid = lax.axis_index("s") * NC + lax.axis_index("c")
    b = wid // wpb
    j = wid % wpb

    pltpu.sync_copy(preds_hbm.at[b], pred_v.at[pl.ds(0, 5)])
    pltpu.sync_copy(tgt_hbm.at[b], tgt_v)
    pltpu.sync_copy(len_hbm, len_v)

    iota = lax.iota(jnp.int32, L)
    scr_v[...] = len_v[...][jnp.full((L,), b)]
    length_b = scr_v[...][0]

    def xform(c, _):
        cx = pred_v[1, c]
        cy = pred_v[2, c]
        w = pred_v[3, c]
        h = pred_v[4, c]
        hw = w * 0.5
        hh = h * 0.5
        x1 = cx - hw
        x2 = cx + hw
        y1 = cy - hh
        y2 = cy + hh
        pred_v[1, c] = x1
        pred_v[2, c] = x2
        pred_v[3, c] = y1
        pred_v[4, c] = y2
        pred_v[5, c] = (x2 - x1) * (y2 - y1)
        return 0

    lax.fori_loop(0, C, xform, 0, unroll=4)

    nk = (length_b - j + (wpb - 1)) // wpb

    def per_target(k, acc):
        t = j + wpb * k
        trow = tgt_v[t]
        s_cx, s_cy, s_w, s_h = trow[1], trow[2], trow[3], trow[4]
        t_x1 = jnp.full((L,), s_cx - s_w * 0.5)
        t_y1 = jnp.full((L,), s_cy - s_h * 0.5)
        t_x2 = jnp.full((L,), s_cx + s_w * 0.5)
        t_y2 = jnp.full((L,), s_cy + s_h * 0.5)
        area_t = (t_x2 - t_x1) * (t_y2 - t_y1)

        def per_chunk(c, carry):
            run_max, run_idx, idxs = carry
            x1 = pred_v[1, c]
            x2 = pred_v[2, c]
            y1 = pred_v[3, c]
            y2 = pred_v[4, c]
            ap = pred_v[5, c]
            ix1 = jnp.maximum(x1, t_x1)
            iy1 = jnp.maximum(y1, t_y1)
            ix2 = jnp.minimum(x2, t_x2)
            iy2 = jnp.minimum(y2, t_y2)
            inter = jnp.maximum(0.0, ix2 - ix1) * jnp.maximum(0.0, iy2 - iy1)
            iou = inter / (ap + area_t - inter + 1e-06)
            upd = iou > run_max
            run_max = jnp.where(upd, iou, run_max)
            run_idx = jnp.where(upd, idxs, run_idx)
            return run_max, run_idx, idxs + L

        init = (jnp.full((L,), -jnp.inf, jnp.float32), jnp.zeros((L,), jnp.int32), iota)
        run_max, run_idx, _ = lax.fori_loop(0, C, per_chunk, init, unroll=4)

        m = run_max
        for s in (8, 4, 2, 1):
            m = jnp.maximum(m, m[iota ^ s])
        cand = jnp.where(run_max == m, run_idx.astype(jnp.float32),
                         jnp.full((L,), float(C * L), jnp.float32))
        for s in (8, 4, 2, 1):
            cand = jnp.minimum(cand, cand[iota ^ s])
        bestv = jnp.where(m > 0.0, cand, 0.0).astype(jnp.int32)
        scr_v[...] = bestv
        best = scr_v[...][0]

        cb = lax.shift_right_logical(best, 4)
        lane = jnp.full((L,), best & (L - 1))
        obj = pred_v[0, cb][lane]
        x1 = pred_v[1, cb][lane]
        x2 = pred_v[2, cb][lane]
        y1 = pred_v[3, cb][lane]
        y2 = pred_v[4, cb][lane]
        vals = (obj, (x1 + x2) * 0.5, (y1 + y2) * 0.5, x2 - x1, y2 - y1)
        loss = jnp.zeros((L,), jnp.float32)
        for r in range(5):
            dv = vals[r] - jnp.full((L,), trow[r])
            loss = loss + dv * dv
        return acc + loss

    acc = lax.fori_loop(0, nk, per_target, jnp.zeros((L,), jnp.float32))
    out_v[...] = jnp.where(iota == 0, acc, 0.0)
    pltpu.sync_copy(out_v, out_hbm.at[wid])


def kernel(predictions, targets, lengths):
    B, N, F = predictions.shape
    C = N // L
    preds_t = jnp.reshape(predictions.transpose(0, 2, 1), (B, F, C, L))
    tgt_pad = jnp.pad(targets, ((0, 0), (0, 0), (0, L - targets.shape[2])))
    len_pad = jnp.pad(lengths.astype(jnp.int32), (0, L - B))

    mesh = plsc.VectorSubcoreMesh(core_axis_name="c", subcore_axis_name="s",
                                  num_cores=NC, num_subcores=NS)
    run = pl.kernel(
        _sc_loss,
        out_type=jax.ShapeDtypeStruct((NW, L), jnp.float32),
        mesh=mesh,
        compiler_params=pltpu.CompilerParams(use_tc_tiling_on_sc=False),
        scratch_types=[
            pltpu.VMEM((F + 1, C, L), jnp.float32),
            pltpu.VMEM((targets.shape[1], L), jnp.float32),
            pltpu.VMEM((L,), jnp.int32),
            pltpu.VMEM((L,), jnp.float32),
            pltpu.VMEM((L,), jnp.int32),
        ],
    )
    partial = run(preds_t, tgt_pad, len_pad)
    return jnp.sum(partial) / B
